# baseline (speedup 1.0000x reference)
import jax
import jax.numpy as jnp
from jax import lax
from jax.experimental import pallas as pl
from jax.experimental.pallas import tpu as pltpu
from jax.experimental.pallas import tpu_sc as plsc

N = 10000
DEG = 32
D = 128

BLOCK = 400

K_SC = 1600
NW = 32
NPW = K_SC // NW
NB = 5
NBLK = NPW // NB
G = D // 16
OUT_BYTES = NB * D * 4


def _sc_sum_body(neigh_hbm, out_hbm, buf, obuf, sem_in, sem_out):
    wid = lax.axis_index("s") * 2 + lax.axis_index("c")
    base = wid * NPW

    def in_copy(j, slot):
        return pltpu.make_async_copy(
            neigh_hbm.at[pl.ds(base + j * NB, NB)], buf.at[slot], sem_in
        )

    def out_copy(j, slot):
        return pltpu.make_async_copy(
            obuf.at[slot], out_hbm.at[pl.ds((base + j * NB) * D, NB * D)],
            sem_out,
        )

    in_copy(0, 0).start()

    def process(j, slot):
        in_copy(j, slot).wait()

        @pl.when(j + 1 < NBLK)
        def _():
            in_copy(j + 1, 1 - slot).start()

        @pl.when(j >= 2)
        def _():
            out_copy(j, slot).wait()

        def node(i, _):
            accs = [buf[slot, i, 0, pl.ds(g * 16, 16)] for g in range(G)]
            for k in range(1, DEG):
                for g in range(G):
                    accs[g] = accs[g] + buf[slot, i, k, pl.ds(g * 16, 16)]
            for g in range(G):
                obuf[slot, pl.ds(i * D + g * 16, 16)] = accs[g]
            return 0

        lax.fori_loop(0, NB, node, 0)
        out_copy(j, slot).start()

    def outer(jj, _):
        process(2 * jj, 0)
        process(2 * jj + 1, 1)
        return 0

    lax.fori_loop(0, NBLK // 2, outer, 0)
    out_copy(NBLK - 2, 0).wait()
    out_copy(NBLK - 1, 1).wait()


_sc_sum = pl.kernel(
    _sc_sum_body,
    out_type=jax.ShapeDtypeStruct((K_SC * D,), jnp.float32),
    mesh=plsc.VectorSubcoreMesh(core_axis_name="c", subcore_axis_name="s"),
    scratch_types=[
        pltpu.VMEM((2, NB, DEG, D), jnp.float32),
        pltpu.VMEM((2, NB * D), jnp.float32),
        pltpu.SemaphoreType.DMA,
        pltpu.SemaphoreType.DMA,
    ],
)


def _fused_body(src_ref, neigh_ref, w_ref, b_ref, out_ref):
    agg = jnp.sum(neigh_ref[...], axis=1) * (1.0 / DEG)
    h = jnp.dot(agg, w_ref[...], preferred_element_type=jnp.float32)
    h += jnp.dot(src_ref[...], b_ref[...], preferred_element_type=jnp.float32)
    out_ref[...] = jnp.maximum(h, 0.0)


def _combine_body(full_ref, agg_ref, src_ref, w_ref, b_ref, out_ref):
    del full_ref
    h = jnp.dot(agg_ref[...] * (1.0 / DEG), w_ref[...],
                preferred_element_type=jnp.float32)
    h += jnp.dot(src_ref[...], b_ref[...], preferred_element_type=jnp.float32)
    out_ref[...] = jnp.maximum(h, 0.0)


_SC_BLOCKS = K_SC // BLOCK


def kernel(src_node_features, neighbor_node_features, W_agg, b):
    agg_sc = _sc_sum(neighbor_node_features).reshape(K_SC, D)

    out_rest = pl.pallas_call(
        _fused_body,
        grid=((N - K_SC) // BLOCK,),
        in_specs=[
            pl.BlockSpec((BLOCK, D), lambda i: (i + _SC_BLOCKS, 0)),
            pl.BlockSpec((BLOCK, DEG, D), lambda i: (i + _SC_BLOCKS, 0, 0)),
            pl.BlockSpec((D, D), lambda i: (0, 0)),
            pl.BlockSpec((D, D), lambda i: (0, 0)),
        ],
        out_specs=pl.BlockSpec((BLOCK, D), lambda i: (i + _SC_BLOCKS, 0)),
        out_shape=jax.ShapeDtypeStruct((N, D), jnp.float32),
    )(src_node_features, neighbor_node_features, W_agg, b)

    out = pl.pallas_call(
        _combine_body,
        grid=(_SC_BLOCKS,),
        in_specs=[
            pl.BlockSpec(memory_space=pl.ANY),
            pl.BlockSpec((BLOCK, D), lambda i: (i, 0)),
            pl.BlockSpec((BLOCK, D), lambda i: (i, 0)),
            pl.BlockSpec((D, D), lambda i: (0, 0)),
            pl.BlockSpec((D, D), lambda i: (0, 0)),
        ],
        out_specs=pl.BlockSpec((BLOCK, D), lambda i: (i, 0)),
        out_shape=jax.ShapeDtypeStruct((N, D), jnp.float32),
        input_output_aliases={0: 0},
    )(out_rest, agg_sc, src_node_features, W_agg, b)

    return out


# speedup vs baseline: 1.3776x; 1.3776x over previous
import jax
import jax.numpy as jnp
from jax.experimental import pallas as pl

N = 10000
DEG = 32
D = 128
BLOCK = 400


def _fused_body(src_ref, neigh_ref, w_ref, b_ref, out_ref):
    agg = jnp.sum(neigh_ref[...], axis=1) * (1.0 / DEG)
    h = jnp.dot(agg, w_ref[...], preferred_element_type=jnp.float32)
    h += jnp.dot(src_ref[...], b_ref[...], preferred_element_type=jnp.float32)
    out_ref[...] = jnp.maximum(h, 0.0)


def kernel(src_node_features, neighbor_node_features, W_agg, b):
    grid = N // BLOCK
    return pl.pallas_call(
        _fused_body,
        grid=(grid,),
        in_specs=[
            pl.BlockSpec((BLOCK, D), lambda i: (i, 0)),
            pl.BlockSpec((BLOCK, DEG, D), lambda i: (i, 0, 0)),
            pl.BlockSpec((D, D), lambda i: (0, 0)),
            pl.BlockSpec((D, D), lambda i: (0, 0)),
        ],
        out_specs=pl.BlockSpec((BLOCK, D), lambda i: (i, 0)),
        out_shape=jax.ShapeDtypeStruct((N, D), jnp.float32),
    )(src_node_features, neighbor_node_features, W_agg, b)
